# baseline (speedup 1.0000x reference)
import jax
import jax.numpy as jnp
from jax import lax
from jax.experimental import pallas as pl
from jax.experimental.pallas import tpu as pltpu
from jax.experimental.pallas import tpu_sc as plsc

N = 10000
E = 320000
D = 128
ED = 16
C = 80
NW = 32
CHUNKS = 125
EPW = C * CHUNKS
EP = NW * EPW
SR = 624


def _ab_body(x_ref, ws_ref, wd_ref, a_ref, b_ref):
    x = x_ref[...]
    a_ref[...] = jnp.dot(x, ws_ref[...], preferred_element_type=jnp.float32)
    b_ref[...] = jnp.dot(x, wd_ref[...], preferred_element_type=jnp.float32)


def _ce_body(e_ref, we_ref, b_ref, c_ref):
    c_ref[...] = (
        jnp.dot(e_ref[...], we_ref[...], preferred_element_type=jnp.float32)
        + b_ref[...]
    ).astype(jnp.bfloat16)


def _sc_body(a_hbm, b_hbm, ce_hbm, idx_hbm, parts_hbm, degp_hbm,
             agg, ixv, di1, a_v, b_v, c_v, h_v, degl,
             sem_a, sem_b, sem_c, sem_s, sem_i):
    cid = lax.axis_index("c")
    sid = lax.axis_index("s")
    wid = sid * 2 + cid
    wbase = wid * EPW
    wchunk = wid * CHUNKS

    def zrow(i, carry):
        for j in range(D // 16):
            h_v[i, pl.ds(16 * j, 16)] = jnp.zeros((16,), jnp.float32)
        return carry

    lax.fori_loop(0, C, zrow, 0)

    def zdeg(i, carry):
        degl[pl.ds(16 * i, 16)] = jnp.zeros((16,), jnp.float32)
        return carry

    lax.fori_loop(0, N // 16, zdeg, 0)
    soff = pl.multiple_of(sid * SR, 8)
    for r in range(SR // C):
        pltpu.sync_copy(h_v, agg.at[pl.ds(soff + r * C, C)])
    pltpu.sync_copy(h_v.at[pl.ds(0, SR - (SR // C) * C)],
                    agg.at[pl.ds(soff + (SR // C) * C, SR - (SR // C) * C)])

    @pl.when(sid == 15)
    def _():
        pltpu.sync_copy(h_v.at[pl.ds(0, N - 16 * SR)],
                        agg.at[pl.ds(16 * SR, N - 16 * SR)])

    plsc.subcore_barrier()

    ones16 = jnp.full((16,), 1.0, jnp.float32)
    iota16 = lax.iota(jnp.int32, 16)

    def stage_scatter_idx():
        for k in range(C // 16):
            di1[pl.ds(16 * k, 16)] = ixv[1, pl.ds(16 * k, 16)]

    pltpu.async_copy(idx_hbm.at[wchunk], ixv, sem_i)
    pltpu.make_async_copy(idx_hbm.at[wchunk], ixv, sem_i).wait()
    stage_scatter_idx()
    pltpu.async_copy(h_v, agg.at[di1], sem_s, add=True)

    def chunk(g, carry):
        base2 = pl.multiple_of(wbase // 2 + g * (C // 2), 8)
        pltpu.async_copy(a_hbm.at[ixv.at[0]], a_v, sem_a)
        pltpu.async_copy(b_hbm.at[ixv.at[1]], b_v, sem_b)
        pltpu.async_copy(ce_hbm.at[pl.ds(base2, C // 2)], c_v, sem_c)

        for k in range(C // 16):
            idx16 = ixv[1, pl.ds(16 * k, 16)]
            plsc.addupdate_scatter(degl, [idx16], ones16)

        pltpu.make_async_copy(a_hbm.at[ixv.at[0]], a_v, sem_a).wait()
        pltpu.make_async_copy(b_hbm.at[ixv.at[1]], b_v, sem_b).wait()
        pltpu.make_async_copy(ce_hbm.at[pl.ds(base2, C // 2)], c_v, sem_c).wait()
        pltpu.make_async_copy(h_v, agg.at[di1], sem_s).wait()
        stage_scatter_idx()
        pltpu.async_copy(idx_hbm.at[wchunk + g + 1], ixv, sem_i)

        def rowpair(r, rcarry):
            for half in range(2):
                i = 2 * r + half
                for j in range(D // 32):
                    cw = plsc.bitcast(
                        c_v[r, pl.ds(64 * half + 16 * j, 16)], jnp.bfloat16)
                    c0, c1 = plsc.unpack(
                        cw, format=plsc.PackFormat.INTERLEAVED,
                        preferred_element_type=jnp.float32)
                    s0 = pl.ds(32 * j, 16)
                    s1 = pl.ds(32 * j + 16, 16)
                    h_v[i, s0] = jnp.maximum(a_v[i, s0] + b_v[i, s0] + c0, 0.0)
                    h_v[i, s1] = jnp.maximum(a_v[i, s1] + b_v[i, s1] + c1, 0.0)
            return rcarry

        lax.fori_loop(0, C // 2, rowpair, 0)
        pltpu.async_copy(h_v, agg.at[di1], sem_s, add=True)
        pltpu.make_async_copy(idx_hbm.at[wchunk + g + 1], ixv, sem_i).wait()
        return carry

    lax.fori_loop(0, CHUNKS, chunk, 0)
    pltpu.make_async_copy(h_v, agg.at[di1], sem_s).wait()

    pltpu.sync_copy(degl, degp_hbm.at[wid, 0])
    plsc.subcore_barrier()
    pltpu.sync_copy(
        agg.at[pl.ds(soff, SR)],
        parts_hbm.at[cid, pl.ds(soff, SR)],
    )

    @pl.when(sid == 15)
    def _():
        pltpu.sync_copy(
            agg.at[pl.ds(16 * SR, N - 16 * SR)],
            parts_hbm.at[cid, pl.ds(16 * SR, N - 16 * SR)],
        )


def _deg_body(ones_ref, d_ref, o_ref):
    o_ref[...] = jnp.dot(ones_ref[...], d_ref[...],
                         preferred_element_type=jnp.float32)


def _post_body(x_ref, p0_ref, p1_ref, deg_ref, wm2_ref, bm2_ref,
               wu1x_ref, wu1a_ref, bu1_ref, wu2_ref, bu2_ref, g_ref, bt_ref,
               o_ref):
    aggh = p0_ref[...] + p1_ref[...]
    deg = deg_ref[...]
    x = x_ref[...]
    aggregated = (
        jnp.dot(aggh, wm2_ref[...], preferred_element_type=jnp.float32)
        + deg * bm2_ref[...]
    )
    h2 = jnp.maximum(
        jnp.dot(x, wu1x_ref[...], preferred_element_type=jnp.float32)
        + jnp.dot(aggregated, wu1a_ref[...], preferred_element_type=jnp.float32)
        + bu1_ref[...],
        0.0,
    )
    y = x + jnp.dot(h2, wu2_ref[...], preferred_element_type=jnp.float32) + bu2_ref[...]
    mu = jnp.mean(y, axis=1, keepdims=True)
    var = jnp.mean((y - mu) ** 2, axis=1, keepdims=True)
    o_ref[...] = (y - mu) * lax.rsqrt(var + 1e-5) * g_ref[...] + bt_ref[...]


def kernel(node_features, edge_index, edge_features, Wm1, bm1, Wm2, bm2,
           Wu1, bu1, Wu2, bu2, gamma, beta):
    f32 = jnp.float32
    sidx = edge_index[0].astype(jnp.int32)
    didx = edge_index[1].astype(jnp.int32)
    idx3 = jnp.concatenate([
        jnp.stack([sidx.reshape(-1, C), didx.reshape(-1, C)], axis=1),
        jnp.zeros((1, 2, C), jnp.int32),
    ])

    pb = jnp.stack([jnp.arange(16), jnp.arange(16) + 16], axis=1).reshape(32)
    perm = (pb[None, :] + 32 * jnp.arange(D // 32)[:, None]).reshape(D)
    ws_t = Wm1[:, :D].T
    wd_t = Wm1[:, D:2 * D].T
    we_t = Wm1[:, 2 * D:].T[:, perm]
    bm1_p = bm1[perm]

    a_tab, b_tab = pl.pallas_call(
        _ab_body,
        grid=(N // 1000,),
        in_specs=[
            pl.BlockSpec((1000, D), lambda i: (i, 0)),
            pl.BlockSpec((D, D), lambda i: (0, 0)),
            pl.BlockSpec((D, D), lambda i: (0, 0)),
        ],
        out_specs=[
            pl.BlockSpec((1000, D), lambda i: (i, 0)),
            pl.BlockSpec((1000, D), lambda i: (i, 0)),
        ],
        out_shape=[
            jax.ShapeDtypeStruct((N, D), f32),
            jax.ShapeDtypeStruct((N, D), f32),
        ],
    )(node_features, ws_t, wd_t)

    EB = 3200
    ce_tab = pl.pallas_call(
        _ce_body,
        grid=(EP // EB,),
        in_specs=[
            pl.BlockSpec((EB, ED), lambda i: (i, 0)),
            pl.BlockSpec((ED, D), lambda i: (0, 0)),
            pl.BlockSpec((1, D), lambda i: (0, 0)),
        ],
        out_specs=pl.BlockSpec((EB, D), lambda i: (i, 0)),
        out_shape=jax.ShapeDtypeStruct((EP, D), jnp.bfloat16),
    )(edge_features, we_t, bm1_p.reshape(1, D))

    parts, degp = pl.kernel(
        _sc_body,
        out_type=(
            jax.ShapeDtypeStruct((2, N, D), f32),
            jax.ShapeDtypeStruct((NW, 1, N), f32),
        ),
        mesh=plsc.VectorSubcoreMesh(core_axis_name="c", subcore_axis_name="s"),
        compiler_params=pltpu.CompilerParams(needs_layout_passes=False),
        scratch_types=[
            pltpu.VMEM_SHARED((N, D), f32),
            pltpu.VMEM((2, C), jnp.int32),
            pltpu.VMEM((C,), jnp.int32),
            pltpu.VMEM((C, D), f32),
            pltpu.VMEM((C, D), f32),
            pltpu.VMEM((C // 2, D), f32),
            pltpu.VMEM((C, D), f32),
            pltpu.VMEM((N,), f32),
            pltpu.SemaphoreType.DMA,
            pltpu.SemaphoreType.DMA,
            pltpu.SemaphoreType.DMA,
            pltpu.SemaphoreType.DMA,
            pltpu.SemaphoreType.DMA,
        ],
    )(a_tab, b_tab,
      lax.bitcast_convert_type(ce_tab.reshape(EP // 2, D, 2), f32),
      idx3)

    degsum = pl.pallas_call(
        _deg_body,
        grid=(1,),
        in_specs=[
            pl.BlockSpec((1, NW), lambda i: (0, 0)),
            pl.BlockSpec((NW, N), lambda i: (0, 0)),
        ],
        out_specs=pl.BlockSpec((1, N), lambda i: (0, 0)),
        out_shape=jax.ShapeDtypeStruct((1, N), f32),
    )(jnp.ones((1, NW), f32), degp.reshape(NW, N))
    deg_col = degsum.reshape(N, 1)

    RB = 1000
    out = pl.pallas_call(
        _post_body,
        grid=(N // RB,),
        in_specs=[
            pl.BlockSpec((RB, D), lambda i: (i, 0)),
            pl.BlockSpec((RB, D), lambda i: (i, 0)),
            pl.BlockSpec((RB, D), lambda i: (i, 0)),
            pl.BlockSpec((RB, 1), lambda i: (i, 0)),
            pl.BlockSpec((D, D), lambda i: (0, 0)),
            pl.BlockSpec((1, D), lambda i: (0, 0)),
            pl.BlockSpec((D, D), lambda i: (0, 0)),
            pl.BlockSpec((D, D), lambda i: (0, 0)),
            pl.BlockSpec((1, D), lambda i: (0, 0)),
            pl.BlockSpec((D, D), lambda i: (0, 0)),
            pl.BlockSpec((1, D), lambda i: (0, 0)),
            pl.BlockSpec((1, D), lambda i: (0, 0)),
            pl.BlockSpec((1, D), lambda i: (0, 0)),
        ],
        out_specs=pl.BlockSpec((RB, D), lambda i: (i, 0)),
        out_shape=jax.ShapeDtypeStruct((N, D), f32),
    )(node_features, parts[0], parts[1], deg_col, Wm2.T, bm2.reshape(1, D),
      Wu1[:, :D].T, Wu1[:, D:].T, bu1.reshape(1, D), Wu2.T,
      bu2.reshape(1, D), gamma.reshape(1, D), beta.reshape(1, D))
    return out


# speedup vs baseline: 1.0001x; 1.0001x over previous
import jax
import jax.numpy as jnp
from jax import lax
from jax.experimental import pallas as pl
from jax.experimental.pallas import tpu as pltpu
from jax.experimental.pallas import tpu_sc as plsc

N = 10000
E = 320000
D = 128
ED = 16
C = 80
NW = 32
CHUNKS = 125
EPW = C * CHUNKS
EP = NW * EPW
SR = 624


def _ab_body(x_ref, ws_ref, wd_ref, a_ref, b_ref):
    x = x_ref[...]
    a_ref[...] = jnp.dot(x, ws_ref[...], preferred_element_type=jnp.float32)
    b_ref[...] = jnp.dot(x, wd_ref[...], preferred_element_type=jnp.float32)


def _ce_body(e_ref, we_ref, b_ref, c_ref):
    c_ref[...] = (
        jnp.dot(e_ref[...], we_ref[...], preferred_element_type=jnp.float32)
        + b_ref[...]
    ).astype(jnp.bfloat16)


def _sc_body(a_hbm, b_hbm, ce_hbm, sidx_hbm, didx_hbm, parts_hbm, degp_hbm,
             agg, si0, di0, di1, a_v, b_v, c_v, h_v, degl,
             sem_a, sem_b, sem_c, sem_s, sem_i):
    cid = lax.axis_index("c")
    sid = lax.axis_index("s")
    wid = sid * 2 + cid
    wbase = wid * EPW

    def zrow(i, carry):
        for j in range(D // 16):
            h_v[i, pl.ds(16 * j, 16)] = jnp.zeros((16,), jnp.float32)
        return carry

    lax.fori_loop(0, C, zrow, 0)

    def zdeg(i, carry):
        degl[pl.ds(16 * i, 16)] = jnp.zeros((16,), jnp.float32)
        return carry

    lax.fori_loop(0, N // 16, zdeg, 0)
    soff = pl.multiple_of(sid * SR, 8)
    for r in range(SR // C):
        pltpu.sync_copy(h_v, agg.at[pl.ds(soff + r * C, C)])
    pltpu.sync_copy(h_v.at[pl.ds(0, SR - (SR // C) * C)],
                    agg.at[pl.ds(soff + (SR // C) * C, SR - (SR // C) * C)])

    @pl.when(sid == 15)
    def _():
        pltpu.sync_copy(h_v.at[pl.ds(0, N - 16 * SR)],
                        agg.at[pl.ds(16 * SR, N - 16 * SR)])

    plsc.subcore_barrier()

    ones16 = jnp.full((16,), 1.0, jnp.float32)
    iota16 = lax.iota(jnp.int32, 16)

    def stage_scatter_idx():
        for k in range(C // 16):
            di1[pl.ds(16 * k, 16)] = di0[pl.ds(16 * k, 16)]

    def issue_idx(base):
        pltpu.async_copy(sidx_hbm.at[pl.ds(base, C)], si0, sem_i)
        pltpu.async_copy(didx_hbm.at[pl.ds(base, C)], di0, sem_i)

    def wait_idx(base):
        pltpu.make_async_copy(sidx_hbm.at[pl.ds(base, C)], si0, sem_i).wait()
        pltpu.make_async_copy(didx_hbm.at[pl.ds(base, C)], di0, sem_i).wait()

    issue_idx(wbase)
    wait_idx(wbase)
    stage_scatter_idx()
    pltpu.async_copy(h_v, agg.at[di1], sem_s, add=True)

    def chunk(g, carry):
        base2 = pl.multiple_of(wbase // 2 + g * (C // 2), 8)
        pltpu.async_copy(a_hbm.at[si0], a_v, sem_a)
        pltpu.async_copy(b_hbm.at[di0], b_v, sem_b)
        pltpu.async_copy(ce_hbm.at[pl.ds(base2, C // 2)], c_v, sem_c)

        for k in range(C // 16):
            idx16 = di0[pl.ds(16 * k, 16)]
            plsc.addupdate_scatter(degl, [idx16], ones16)

        pltpu.make_async_copy(a_hbm.at[si0], a_v, sem_a).wait()
        pltpu.make_async_copy(b_hbm.at[di0], b_v, sem_b).wait()
        pltpu.make_async_copy(ce_hbm.at[pl.ds(base2, C // 2)], c_v, sem_c).wait()
        pltpu.make_async_copy(h_v, agg.at[di1], sem_s).wait()
        stage_scatter_idx()
        base = wbase + g * C
        issue_idx(base + C)

        def rowpair(r, rcarry):
            for half in range(2):
                i = 2 * r + half
                for j in range(D // 32):
                    cw = plsc.bitcast(
                        c_v[r, pl.ds(64 * half + 16 * j, 16)], jnp.bfloat16)
                    c0, c1 = plsc.unpack(
                        cw, format=plsc.PackFormat.INTERLEAVED,
                        preferred_element_type=jnp.float32)
                    s0 = pl.ds(32 * j, 16)
                    s1 = pl.ds(32 * j + 16, 16)
                    h_v[i, s0] = jnp.maximum(a_v[i, s0] + b_v[i, s0] + c0, 0.0)
                    h_v[i, s1] = jnp.maximum(a_v[i, s1] + b_v[i, s1] + c1, 0.0)
            return rcarry

        lax.fori_loop(0, C // 2, rowpair, 0)
        pltpu.async_copy(h_v, agg.at[di1], sem_s, add=True)
        wait_idx(base + C)
        return carry

    lax.fori_loop(0, CHUNKS, chunk, 0)
    pltpu.make_async_copy(h_v, agg.at[di1], sem_s).wait()

    pltpu.sync_copy(degl, degp_hbm.at[wid, 0])
    plsc.subcore_barrier()
    pltpu.sync_copy(
        agg.at[pl.ds(soff, SR)],
        parts_hbm.at[cid, pl.ds(soff, SR)],
    )

    @pl.when(sid == 15)
    def _():
        pltpu.sync_copy(
            agg.at[pl.ds(16 * SR, N - 16 * SR)],
            parts_hbm.at[cid, pl.ds(16 * SR, N - 16 * SR)],
        )


def _deg_body(ones_ref, d_ref, o_ref):
    o_ref[...] = jnp.dot(ones_ref[...], d_ref[...],
                         preferred_element_type=jnp.float32)


def _post_body(x_ref, p0_ref, p1_ref, deg_ref, wm2_ref, bm2_ref,
               wu1x_ref, wu1a_ref, bu1_ref, wu2_ref, bu2_ref, g_ref, bt_ref,
               o_ref):
    aggh = p0_ref[...] + p1_ref[...]
    deg = deg_ref[...]
    x = x_ref[...]
    aggregated = (
        jnp.dot(aggh, wm2_ref[...], preferred_element_type=jnp.float32)
        + deg * bm2_ref[...]
    )
    h2 = jnp.maximum(
        jnp.dot(x, wu1x_ref[...], preferred_element_type=jnp.float32)
        + jnp.dot(aggregated, wu1a_ref[...], preferred_element_type=jnp.float32)
        + bu1_ref[...],
        0.0,
    )
    y = x + jnp.dot(h2, wu2_ref[...], preferred_element_type=jnp.float32) + bu2_ref[...]
    mu = jnp.mean(y, axis=1, keepdims=True)
    var = jnp.mean((y - mu) ** 2, axis=1, keepdims=True)
    o_ref[...] = (y - mu) * lax.rsqrt(var + 1e-5) * g_ref[...] + bt_ref[...]


def kernel(node_features, edge_index, edge_features, Wm1, bm1, Wm2, bm2,
           Wu1, bu1, Wu2, bu2, gamma, beta):
    f32 = jnp.float32
    slack = jnp.zeros((C,), jnp.int32)
    sidx = jnp.concatenate([edge_index[0].astype(jnp.int32), slack])
    didx = jnp.concatenate([edge_index[1].astype(jnp.int32), slack])

    pb = jnp.stack([jnp.arange(16), jnp.arange(16) + 16], axis=1).reshape(32)
    perm = (pb[None, :] + 32 * jnp.arange(D // 32)[:, None]).reshape(D)
    ws_t = Wm1[:, :D].T
    wd_t = Wm1[:, D:2 * D].T
    we_t = Wm1[:, 2 * D:].T[:, perm]
    bm1_p = bm1[perm]

    a_tab, b_tab = pl.pallas_call(
        _ab_body,
        grid=(N // 1000,),
        in_specs=[
            pl.BlockSpec((1000, D), lambda i: (i, 0)),
            pl.BlockSpec((D, D), lambda i: (0, 0)),
            pl.BlockSpec((D, D), lambda i: (0, 0)),
        ],
        out_specs=[
            pl.BlockSpec((1000, D), lambda i: (i, 0)),
            pl.BlockSpec((1000, D), lambda i: (i, 0)),
        ],
        out_shape=[
            jax.ShapeDtypeStruct((N, D), f32),
            jax.ShapeDtypeStruct((N, D), f32),
        ],
    )(node_features, ws_t, wd_t)

    EB = 3200
    ce_tab = pl.pallas_call(
        _ce_body,
        grid=(EP // EB,),
        in_specs=[
            pl.BlockSpec((EB, ED), lambda i: (i, 0)),
            pl.BlockSpec((ED, D), lambda i: (0, 0)),
            pl.BlockSpec((1, D), lambda i: (0, 0)),
        ],
        out_specs=pl.BlockSpec((EB, D), lambda i: (i, 0)),
        out_shape=jax.ShapeDtypeStruct((EP, D), jnp.bfloat16),
    )(edge_features, we_t, bm1_p.reshape(1, D))

    parts, degp = pl.kernel(
        _sc_body,
        out_type=(
            jax.ShapeDtypeStruct((2, N, D), f32),
            jax.ShapeDtypeStruct((NW, 1, N), f32),
        ),
        mesh=plsc.VectorSubcoreMesh(core_axis_name="c", subcore_axis_name="s"),
        compiler_params=pltpu.CompilerParams(needs_layout_passes=False),
        scratch_types=[
            pltpu.VMEM_SHARED((N, D), f32),
            pltpu.VMEM((C,), jnp.int32),
            pltpu.VMEM((C,), jnp.int32),
            pltpu.VMEM((C,), jnp.int32),
            pltpu.VMEM((C, D), f32),
            pltpu.VMEM((C, D), f32),
            pltpu.VMEM((C // 2, D), f32),
            pltpu.VMEM((C, D), f32),
            pltpu.VMEM((N,), f32),
            pltpu.SemaphoreType.DMA,
            pltpu.SemaphoreType.DMA,
            pltpu.SemaphoreType.DMA,
            pltpu.SemaphoreType.DMA,
            pltpu.SemaphoreType.DMA,
        ],
    )(a_tab, b_tab,
      lax.bitcast_convert_type(ce_tab.reshape(EP // 2, D, 2), f32),
      sidx, didx)

    degsum = pl.pallas_call(
        _deg_body,
        grid=(1,),
        in_specs=[
            pl.BlockSpec((1, NW), lambda i: (0, 0)),
            pl.BlockSpec((NW, N), lambda i: (0, 0)),
        ],
        out_specs=pl.BlockSpec((1, N), lambda i: (0, 0)),
        out_shape=jax.ShapeDtypeStruct((1, N), f32),
    )(jnp.ones((1, NW), f32), degp.reshape(NW, N))
    deg_col = degsum.reshape(N, 1)

    RB = 1000
    out = pl.pallas_call(
        _post_body,
        grid=(N // RB,),
        in_specs=[
            pl.BlockSpec((RB, D), lambda i: (i, 0)),
            pl.BlockSpec((RB, D), lambda i: (i, 0)),
            pl.BlockSpec((RB, D), lambda i: (i, 0)),
            pl.BlockSpec((RB, 1), lambda i: (i, 0)),
            pl.BlockSpec((D, D), lambda i: (0, 0)),
            pl.BlockSpec((1, D), lambda i: (0, 0)),
            pl.BlockSpec((D, D), lambda i: (0, 0)),
            pl.BlockSpec((D, D), lambda i: (0, 0)),
            pl.BlockSpec((1, D), lambda i: (0, 0)),
            pl.BlockSpec((D, D), lambda i: (0, 0)),
            pl.BlockSpec((1, D), lambda i: (0, 0)),
            pl.BlockSpec((1, D), lambda i: (0, 0)),
            pl.BlockSpec((1, D), lambda i: (0, 0)),
        ],
        out_specs=pl.BlockSpec((RB, D), lambda i: (i, 0)),
        out_shape=jax.ShapeDtypeStruct((N, D), f32),
    )(node_features, parts[0], parts[1], deg_col, Wm2.T, bm2.reshape(1, D),
      Wu1[:, :D].T, Wu1[:, D:].T, bu1.reshape(1, D), Wu2.T,
      bu2.reshape(1, D), gamma.reshape(1, D), beta.reshape(1, D))
    return out


# speedup vs baseline: 8.5603x; 8.5590x over previous
import jax
import jax.numpy as jnp
from jax import lax
from jax.experimental import pallas as pl
from jax.experimental.pallas import tpu as pltpu
from jax.experimental.pallas import tpu_sc as plsc

N = 10000
E = 320000
D = 128
ED = 16
C = 80
NW = 32
CHUNKS = 125
EPW = C * CHUNKS
EP = NW * EPW
SR = 624


def _ab_body(x_ref, ws_ref, wd_ref, a_ref, b_ref):
    x = x_ref[...]
    a_ref[...] = jnp.dot(x, ws_ref[...], preferred_element_type=jnp.float32)
    b_ref[...] = jnp.dot(x, wd_ref[...], preferred_element_type=jnp.float32)


def _ce_body(e_ref, we_ref, b_ref, c_ref):
    c_ref[...] = (
        jnp.dot(e_ref[...], we_ref[...], preferred_element_type=jnp.float32)
        + b_ref[...]
    ).astype(jnp.bfloat16)


def _sc_body(a_hbm, b_hbm, ce_hbm, sidx_hbm, didx_hbm, parts_hbm, degp_hbm,
             agg, si0, di0, di1, a_v, b_v, c_v, h_v, degl,
             sem_a, sem_b, sem_c, sem_s, sem_i):
    cid = lax.axis_index("c")
    sid = lax.axis_index("s")
    wid = sid * 2 + cid
    wbase = wid * EPW

    def zrow(i, carry):
        for j in range(D // 16):
            h_v[i, pl.ds(16 * j, 16)] = jnp.zeros((16,), jnp.float32)
        return carry

    lax.fori_loop(0, C, zrow, 0)

    def zdeg(i, carry):
        degl[pl.ds(16 * i, 16)] = jnp.zeros((16,), jnp.float32)
        return carry

    lax.fori_loop(0, N // 16, zdeg, 0)
    soff = pl.multiple_of(sid * SR, 8)
    for r in range(SR // C):
        pltpu.sync_copy(h_v, agg.at[pl.ds(soff + r * C, C)])
    pltpu.sync_copy(h_v.at[pl.ds(0, SR - (SR // C) * C)],
                    agg.at[pl.ds(soff + (SR // C) * C, SR - (SR // C) * C)])

    @pl.when(sid == 15)
    def _():
        pltpu.sync_copy(h_v.at[pl.ds(0, N - 16 * SR)],
                        agg.at[pl.ds(16 * SR, N - 16 * SR)])

    plsc.subcore_barrier()

    ones16 = jnp.full((16,), 1.0, jnp.float32)
    iota16 = lax.iota(jnp.int32, 16)

    def stage_scatter_idx():
        for k in range(C // 16):
            di1[pl.ds(16 * k, 16)] = di0[pl.ds(16 * k, 16)]

    def issue_idx(base):
        pltpu.async_copy(sidx_hbm.at[pl.ds(base, C)], si0, sem_i)
        pltpu.async_copy(didx_hbm.at[pl.ds(base, C)], di0, sem_i)

    def wait_idx(base):
        pltpu.make_async_copy(sidx_hbm.at[pl.ds(base, C)], si0, sem_i).wait()
        pltpu.make_async_copy(didx_hbm.at[pl.ds(base, C)], di0, sem_i).wait()

    issue_idx(wbase)
    wait_idx(wbase)
    stage_scatter_idx()
    pltpu.async_copy(h_v, agg.at[di1], sem_s, add=True)

    def chunk(g, carry):
        basew = pl.multiple_of((wbase + g * C) * (D // 2), 8)
        pltpu.async_copy(a_hbm.at[si0], a_v, sem_a)
        pltpu.async_copy(b_hbm.at[di0], b_v, sem_b)
        pltpu.async_copy(ce_hbm.at[pl.ds(basew, C * (D // 2))], c_v, sem_c)

        for k in range(C // 16):
            idx16 = di0[pl.ds(16 * k, 16)]
            plsc.addupdate_scatter(degl, [idx16], ones16)

        pltpu.make_async_copy(a_hbm.at[si0], a_v, sem_a).wait()
        pltpu.make_async_copy(b_hbm.at[di0], b_v, sem_b).wait()
        pltpu.make_async_copy(
            ce_hbm.at[pl.ds(basew, C * (D // 2))], c_v, sem_c).wait()
        pltpu.make_async_copy(h_v, agg.at[di1], sem_s).wait()
        stage_scatter_idx()
        base = wbase + g * C
        issue_idx(base + C)

        def row(i, rcarry):
            for j in range(D // 32):
                cw = plsc.bitcast(
                    c_v[pl.ds(i * (D // 2) + 16 * j, 16)], jnp.bfloat16)
                c0, c1 = plsc.unpack(
                    cw, format=plsc.PackFormat.INTERLEAVED,
                    preferred_element_type=jnp.float32)
                s0 = pl.ds(32 * j, 16)
                s1 = pl.ds(32 * j + 16, 16)
                h_v[i, s0] = jnp.maximum(a_v[i, s0] + b_v[i, s0] + c0, 0.0)
                h_v[i, s1] = jnp.maximum(a_v[i, s1] + b_v[i, s1] + c1, 0.0)
            return rcarry

        lax.fori_loop(0, C, row, 0)
        pltpu.async_copy(h_v, agg.at[di1], sem_s, add=True)
        wait_idx(base + C)
        return carry

    lax.fori_loop(0, CHUNKS, chunk, 0)
    pltpu.make_async_copy(h_v, agg.at[di1], sem_s).wait()

    pltpu.sync_copy(degl, degp_hbm.at[wid, 0])
    plsc.subcore_barrier()
    pltpu.sync_copy(
        agg.at[pl.ds(soff, SR)],
        parts_hbm.at[cid, pl.ds(soff, SR)],
    )

    @pl.when(sid == 15)
    def _():
        pltpu.sync_copy(
            agg.at[pl.ds(16 * SR, N - 16 * SR)],
            parts_hbm.at[cid, pl.ds(16 * SR, N - 16 * SR)],
        )


def _deg_body(ones_ref, d_ref, o_ref):
    o_ref[...] = jnp.dot(ones_ref[...], d_ref[...],
                         preferred_element_type=jnp.float32)


def _post_body(x_ref, p0_ref, p1_ref, deg_ref, wm2_ref, bm2_ref,
               wu1x_ref, wu1a_ref, bu1_ref, wu2_ref, bu2_ref, g_ref, bt_ref,
               o_ref):
    aggh = p0_ref[...] + p1_ref[...]
    deg = deg_ref[...]
    x = x_ref[...]
    aggregated = (
        jnp.dot(aggh, wm2_ref[...], preferred_element_type=jnp.float32)
        + deg * bm2_ref[...]
    )
    h2 = jnp.maximum(
        jnp.dot(x, wu1x_ref[...], preferred_element_type=jnp.float32)
        + jnp.dot(aggregated, wu1a_ref[...], preferred_element_type=jnp.float32)
        + bu1_ref[...],
        0.0,
    )
    y = x + jnp.dot(h2, wu2_ref[...], preferred_element_type=jnp.float32) + bu2_ref[...]
    mu = jnp.mean(y, axis=1, keepdims=True)
    var = jnp.mean((y - mu) ** 2, axis=1, keepdims=True)
    o_ref[...] = (y - mu) * lax.rsqrt(var + 1e-5) * g_ref[...] + bt_ref[...]


def kernel(node_features, edge_index, edge_features, Wm1, bm1, Wm2, bm2,
           Wu1, bu1, Wu2, bu2, gamma, beta):
    f32 = jnp.float32
    slack = jnp.zeros((C,), jnp.int32)
    sidx = jnp.concatenate([edge_index[0].astype(jnp.int32), slack])
    didx = jnp.concatenate([edge_index[1].astype(jnp.int32), slack])

    pb = jnp.stack([jnp.arange(16), jnp.arange(16) + 16], axis=1).reshape(32)
    perm = (pb[None, :] + 32 * jnp.arange(D // 32)[:, None]).reshape(D)
    ws_t = Wm1[:, :D].T
    wd_t = Wm1[:, D:2 * D].T
    we_t = Wm1[:, 2 * D:].T[:, perm]
    bm1_p = bm1[perm]

    a_tab, b_tab = pl.pallas_call(
        _ab_body,
        grid=(N // 1000,),
        in_specs=[
            pl.BlockSpec((1000, D), lambda i: (i, 0)),
            pl.BlockSpec((D, D), lambda i: (0, 0)),
            pl.BlockSpec((D, D), lambda i: (0, 0)),
        ],
        out_specs=[
            pl.BlockSpec((1000, D), lambda i: (i, 0)),
            pl.BlockSpec((1000, D), lambda i: (i, 0)),
        ],
        out_shape=[
            jax.ShapeDtypeStruct((N, D), f32),
            jax.ShapeDtypeStruct((N, D), f32),
        ],
    )(node_features, ws_t, wd_t)

    EB = 3200
    ce_tab = pl.pallas_call(
        _ce_body,
        grid=(EP // EB,),
        in_specs=[
            pl.BlockSpec((EB, ED), lambda i: (i, 0)),
            pl.BlockSpec((ED, D), lambda i: (0, 0)),
            pl.BlockSpec((1, D), lambda i: (0, 0)),
        ],
        out_specs=pl.BlockSpec((EB, D), lambda i: (i, 0)),
        out_shape=jax.ShapeDtypeStruct((EP, D), jnp.bfloat16),
    )(edge_features, we_t, bm1_p.reshape(1, D))

    parts, degp = pl.kernel(
        _sc_body,
        out_type=(
            jax.ShapeDtypeStruct((2, N, D), f32),
            jax.ShapeDtypeStruct((NW, 1, N), f32),
        ),
        mesh=plsc.VectorSubcoreMesh(core_axis_name="c", subcore_axis_name="s"),
        compiler_params=pltpu.CompilerParams(needs_layout_passes=False),
        scratch_types=[
            pltpu.VMEM_SHARED((N, D), f32),
            pltpu.VMEM((C,), jnp.int32),
            pltpu.VMEM((C,), jnp.int32),
            pltpu.VMEM((C,), jnp.int32),
            pltpu.VMEM((C, D), f32),
            pltpu.VMEM((C, D), f32),
            pltpu.VMEM((C * (D // 2),), f32),
            pltpu.VMEM((C, D), f32),
            pltpu.VMEM((N,), f32),
            pltpu.SemaphoreType.DMA,
            pltpu.SemaphoreType.DMA,
            pltpu.SemaphoreType.DMA,
            pltpu.SemaphoreType.DMA,
            pltpu.SemaphoreType.DMA,
        ],
    )(a_tab, b_tab,
      lax.bitcast_convert_type(
          ce_tab.reshape(EP, D // 2, 2), f32).reshape(EP * (D // 2)),
      sidx, didx)

    degsum = pl.pallas_call(
        _deg_body,
        grid=(1,),
        in_specs=[
            pl.BlockSpec((1, NW), lambda i: (0, 0)),
            pl.BlockSpec((NW, N), lambda i: (0, 0)),
        ],
        out_specs=pl.BlockSpec((1, N), lambda i: (0, 0)),
        out_shape=jax.ShapeDtypeStruct((1, N), f32),
    )(jnp.ones((1, NW), f32), degp.reshape(NW, N))
    deg_col = degsum.reshape(N, 1)

    RB = 1000
    out = pl.pallas_call(
        _post_body,
        grid=(N // RB,),
        in_specs=[
            pl.BlockSpec((RB, D), lambda i: (i, 0)),
            pl.BlockSpec((RB, D), lambda i: (i, 0)),
            pl.BlockSpec((RB, D), lambda i: (i, 0)),
            pl.BlockSpec((RB, 1), lambda i: (i, 0)),
            pl.BlockSpec((D, D), lambda i: (0, 0)),
            pl.BlockSpec((1, D), lambda i: (0, 0)),
            pl.BlockSpec((D, D), lambda i: (0, 0)),
            pl.BlockSpec((D, D), lambda i: (0, 0)),
            pl.BlockSpec((1, D), lambda i: (0, 0)),
            pl.BlockSpec((D, D), lambda i: (0, 0)),
            pl.BlockSpec((1, D), lambda i: (0, 0)),
            pl.BlockSpec((1, D), lambda i: (0, 0)),
            pl.BlockSpec((1, D), lambda i: (0, 0)),
        ],
        out_specs=pl.BlockSpec((RB, D), lambda i: (i, 0)),
        out_shape=jax.ShapeDtypeStruct((N, D), f32),
    )(node_features, parts[0], parts[1], deg_col, Wm2.T, bm2.reshape(1, D),
      Wu1[:, :D].T, Wu1[:, D:].T, bu1.reshape(1, D), Wu2.T,
      bu2.reshape(1, D), gamma.reshape(1, D), beta.reshape(1, D))
    return out


# speedup vs baseline: 15.6285x; 1.8257x over previous
import jax
import jax.numpy as jnp
from jax import lax
from jax.experimental import pallas as pl
from jax.experimental.pallas import tpu as pltpu
from jax.experimental.pallas import tpu_sc as plsc

N = 10000
E = 320000
D = 128
ED = 16
C = 64
NW = 32
CHUNKS = 158
EPW = C * CHUNKS
EP = NW * EPW
SR = 624


def _ab_body(x_ref, ws_ref, wd_ref, a_ref, b_ref):
    x = x_ref[...]
    a_ref[...] = jnp.dot(x, ws_ref[...], preferred_element_type=jnp.float32)
    b_ref[...] = jnp.dot(x, wd_ref[...], preferred_element_type=jnp.float32)


def _ce_body(e_ref, we_ref, b_ref, c_ref):
    i = pl.program_id(0)
    eb = e_ref.shape[0]
    rows = lax.broadcasted_iota(jnp.int32, (eb, D), 0) + i * eb
    val = (
        jnp.dot(e_ref[...], we_ref[...], preferred_element_type=jnp.float32)
        + b_ref[...]
    )
    c_ref[...] = jnp.where(rows < E, val, -1e9)


def _sc_body(a_hbm, b_hbm, ce_hbm, sidx_hbm, didx_hbm, parts_hbm, degp_hbm,
             agg, si0, di0, di1, a_v, b_v, c_v, h_v, degl,
             sem_a, sem_b, sem_c, sem_s, sem_i):
    cid = lax.axis_index("c")
    sid = lax.axis_index("s")
    wid = sid * 2 + cid
    wbase = wid * EPW

    def zrow(i, carry):
        for j in range(D // 16):
            h_v[i, pl.ds(16 * j, 16)] = jnp.zeros((16,), jnp.float32)
        return carry

    lax.fori_loop(0, C, zrow, 0)

    def zdeg(i, carry):
        degl[pl.ds(16 * i, 16)] = jnp.zeros((16,), jnp.float32)
        return carry

    lax.fori_loop(0, N // 16, zdeg, 0)
    soff = pl.multiple_of(sid * SR, 8)
    for r in range(SR // C):
        pltpu.sync_copy(h_v, agg.at[pl.ds(soff + r * C, C)])
    pltpu.sync_copy(h_v.at[pl.ds(0, SR - (SR // C) * C)],
                    agg.at[pl.ds(soff + (SR // C) * C, SR - (SR // C) * C)])

    @pl.when(sid == 15)
    def _():
        pltpu.sync_copy(h_v.at[pl.ds(0, N - 16 * SR)],
                        agg.at[pl.ds(16 * SR, N - 16 * SR)])

    plsc.subcore_barrier()

    ones16 = jnp.full((16,), 1.0, jnp.float32)
    iota16 = lax.iota(jnp.int32, 16)

    def issue_idx(base):
        pltpu.async_copy(sidx_hbm.at[pl.ds(base, C)], si0, sem_i)
        pltpu.async_copy(didx_hbm.at[pl.ds(base, C)], di0, sem_i)

    def stage_scatter_idx():
        for k in range(C // 16):
            di1[pl.ds(16 * k, 16)] = di0[pl.ds(16 * k, 16)]

    issue_idx(wbase)
    pltpu.make_async_copy(sidx_hbm.at[pl.ds(wbase, C)], si0, sem_i).wait()
    pltpu.make_async_copy(didx_hbm.at[pl.ds(wbase, C)], di0, sem_i).wait()
    stage_scatter_idx()
    pltpu.async_copy(h_v, agg.at[di1], sem_s, add=True)

    def chunk(g, carry):
        base = wbase + g * C
        pltpu.async_copy(a_hbm.at[si0], a_v, sem_a)
        pltpu.async_copy(b_hbm.at[di0], b_v, sem_b)
        pltpu.async_copy(ce_hbm.at[pl.ds(base, C)], c_v, sem_c)

        for k in range(C // 16):
            idx16 = di0[pl.ds(16 * k, 16)]
            eid = base + 16 * k + iota16
            plsc.addupdate_scatter(degl, [idx16], ones16, mask=eid < E)

        pltpu.make_async_copy(a_hbm.at[si0], a_v, sem_a).wait()
        pltpu.make_async_copy(b_hbm.at[di0], b_v, sem_b).wait()
        pltpu.make_async_copy(ce_hbm.at[pl.ds(base, C)], c_v, sem_c).wait()
        pltpu.make_async_copy(h_v, agg.at[di1], sem_s).wait()
        stage_scatter_idx()
        issue_idx(base + C)

        def row(i, rcarry):
            for j in range(D // 16):
                s = pl.ds(16 * j, 16)
                h_v[i, s] = jnp.maximum(a_v[i, s] + b_v[i, s] + c_v[i, s], 0.0)
            return rcarry

        lax.fori_loop(0, C, row, 0)
        pltpu.async_copy(h_v, agg.at[di1], sem_s, add=True)
        pltpu.make_async_copy(sidx_hbm.at[pl.ds(base, C)], si0, sem_i).wait()
        pltpu.make_async_copy(didx_hbm.at[pl.ds(base, C)], di0, sem_i).wait()
        return carry

    lax.fori_loop(0, CHUNKS, chunk, 0)
    pltpu.make_async_copy(h_v, agg.at[di1], sem_s).wait()

    pltpu.sync_copy(degl, degp_hbm.at[wid, 0])
    plsc.subcore_barrier()
    pltpu.sync_copy(
        agg.at[pl.ds(soff, SR)],
        parts_hbm.at[cid, pl.ds(soff, SR)],
    )

    @pl.when(sid == 15)
    def _():
        pltpu.sync_copy(
            agg.at[pl.ds(16 * SR, N - 16 * SR)],
            parts_hbm.at[cid, pl.ds(16 * SR, N - 16 * SR)],
        )


def _deg_body(ones_ref, d_ref, o_ref):
    o_ref[...] = jnp.dot(ones_ref[...], d_ref[...],
                         preferred_element_type=jnp.float32)


def _post_body(x_ref, p0_ref, p1_ref, deg_ref, wm2_ref, bm2_ref,
               wu1x_ref, wu1a_ref, bu1_ref, wu2_ref, bu2_ref, g_ref, bt_ref,
               o_ref):
    aggh = p0_ref[...] + p1_ref[...]
    deg = deg_ref[...]
    x = x_ref[...]
    aggregated = (
        jnp.dot(aggh, wm2_ref[...], preferred_element_type=jnp.float32)
        + deg * bm2_ref[...]
    )
    h2 = jnp.maximum(
        jnp.dot(x, wu1x_ref[...], preferred_element_type=jnp.float32)
        + jnp.dot(aggregated, wu1a_ref[...], preferred_element_type=jnp.float32)
        + bu1_ref[...],
        0.0,
    )
    y = x + jnp.dot(h2, wu2_ref[...], preferred_element_type=jnp.float32) + bu2_ref[...]
    mu = jnp.mean(y, axis=1, keepdims=True)
    var = jnp.mean((y - mu) ** 2, axis=1, keepdims=True)
    o_ref[...] = (y - mu) * lax.rsqrt(var + 1e-5) * g_ref[...] + bt_ref[...]


def kernel(node_features, edge_index, edge_features, Wm1, bm1, Wm2, bm2,
           Wu1, bu1, Wu2, bu2, gamma, beta):
    f32 = jnp.float32
    e_pad = jnp.pad(edge_features, ((0, EP - E), (0, 0)))
    pad_idx = jnp.zeros((EP - E + C,), jnp.int32)
    sidx = jnp.concatenate([edge_index[0].astype(jnp.int32), pad_idx])
    didx = jnp.concatenate([edge_index[1].astype(jnp.int32), pad_idx])

    ws_t = Wm1[:, :D].T
    wd_t = Wm1[:, D:2 * D].T
    we_t = Wm1[:, 2 * D:].T

    a_tab, b_tab = pl.pallas_call(
        _ab_body,
        grid=(N // 1000,),
        in_specs=[
            pl.BlockSpec((1000, D), lambda i: (i, 0)),
            pl.BlockSpec((D, D), lambda i: (0, 0)),
            pl.BlockSpec((D, D), lambda i: (0, 0)),
        ],
        out_specs=[
            pl.BlockSpec((1000, D), lambda i: (i, 0)),
            pl.BlockSpec((1000, D), lambda i: (i, 0)),
        ],
        out_shape=[
            jax.ShapeDtypeStruct((N, D), f32),
            jax.ShapeDtypeStruct((N, D), f32),
        ],
    )(node_features, ws_t, wd_t)

    EB = 2048
    ce_tab = pl.pallas_call(
        _ce_body,
        grid=(EP // EB,),
        in_specs=[
            pl.BlockSpec((EB, ED), lambda i: (i, 0)),
            pl.BlockSpec((ED, D), lambda i: (0, 0)),
            pl.BlockSpec((1, D), lambda i: (0, 0)),
        ],
        out_specs=pl.BlockSpec((EB, D), lambda i: (i, 0)),
        out_shape=jax.ShapeDtypeStruct((EP, D), f32),
    )(e_pad, we_t, bm1.reshape(1, D))

    parts, degp = pl.kernel(
        _sc_body,
        out_type=(
            jax.ShapeDtypeStruct((2, N, D), f32),
            jax.ShapeDtypeStruct((NW, 1, N), f32),
        ),
        mesh=plsc.VectorSubcoreMesh(core_axis_name="c", subcore_axis_name="s"),
        compiler_params=pltpu.CompilerParams(needs_layout_passes=False),
        scratch_types=[
            pltpu.VMEM_SHARED((N, D), f32),
            pltpu.VMEM((C,), jnp.int32),
            pltpu.VMEM((C,), jnp.int32),
            pltpu.VMEM((C,), jnp.int32),
            pltpu.VMEM((C, D), f32),
            pltpu.VMEM((C, D), f32),
            pltpu.VMEM((C, D), f32),
            pltpu.VMEM((C, D), f32),
            pltpu.VMEM((N,), f32),
            pltpu.SemaphoreType.DMA,
            pltpu.SemaphoreType.DMA,
            pltpu.SemaphoreType.DMA,
            pltpu.SemaphoreType.DMA,
            pltpu.SemaphoreType.DMA,
        ],
    )(a_tab, b_tab, ce_tab, sidx, didx)

    degsum = pl.pallas_call(
        _deg_body,
        grid=(1,),
        in_specs=[
            pl.BlockSpec((1, NW), lambda i: (0, 0)),
            pl.BlockSpec((NW, N), lambda i: (0, 0)),
        ],
        out_specs=pl.BlockSpec((1, N), lambda i: (0, 0)),
        out_shape=jax.ShapeDtypeStruct((1, N), f32),
    )(jnp.ones((1, NW), f32), degp.reshape(NW, N))
    deg_col = degsum.reshape(N, 1)

    RB = 1000
    out = pl.pallas_call(
        _post_body,
        grid=(N // RB,),
        in_specs=[
            pl.BlockSpec((RB, D), lambda i: (i, 0)),
            pl.BlockSpec((RB, D), lambda i: (i, 0)),
            pl.BlockSpec((RB, D), lambda i: (i, 0)),
            pl.BlockSpec((RB, 1), lambda i: (i, 0)),
            pl.BlockSpec((D, D), lambda i: (0, 0)),
            pl.BlockSpec((1, D), lambda i: (0, 0)),
            pl.BlockSpec((D, D), lambda i: (0, 0)),
            pl.BlockSpec((D, D), lambda i: (0, 0)),
            pl.BlockSpec((1, D), lambda i: (0, 0)),
            pl.BlockSpec((D, D), lambda i: (0, 0)),
            pl.BlockSpec((1, D), lambda i: (0, 0)),
            pl.BlockSpec((1, D), lambda i: (0, 0)),
            pl.BlockSpec((1, D), lambda i: (0, 0)),
        ],
        out_specs=pl.BlockSpec((RB, D), lambda i: (i, 0)),
        out_shape=jax.ShapeDtypeStruct((N, D), f32),
    )(node_features, parts[0], parts[1], deg_col, Wm2.T, bm2.reshape(1, D),
      Wu1[:, :D].T, Wu1[:, D:].T, bu1.reshape(1, D), Wu2.T,
      bu2.reshape(1, D), gamma.reshape(1, D), beta.reshape(1, D))
    return out
